# initial kernel scaffold (unmeasured)
import jax
import jax.numpy as jnp
from jax import lax
from jax.experimental import pallas as pl
from jax.experimental.pallas import tpu as pltpu

N_DEV = 4
COMM_DTYPE = jnp.float32


def kernel(x, w_mat, scale_x, scale_w):
    m, _k = x.shape
    _, n = w_mat.shape
    m_per = m // N_DEV

    def body(x_ref, w_ref, sx_ref, sw_ref, out_ref,
             stage_ref, recv_ref, send_sems, recv_sems):
        my = lax.axis_index("i")
        left = lax.rem(my + N_DEV - 1, N_DEV)
        right = lax.rem(my + 1, N_DEV)

        barrier_sem = pltpu.get_barrier_semaphore()
        for nbr in (left, right):
            pl.semaphore_signal(
                barrier_sem, inc=1,
                device_id=(nbr,), device_id_type=pl.DeviceIdType.MESH,
            )
        pl.semaphore_wait(barrier_sem, 2)

        def chunk_partial(c):
            xc = x_ref[pl.ds(c * m_per, m_per), :]
            return jnp.dot(xc, w_ref[...], preferred_element_type=jnp.float32)

        c0 = lax.rem(my + N_DEV - 1, N_DEV)
        stage_ref[0] = chunk_partial(c0).astype(COMM_DTYPE)

        for s in range(N_DEV - 1):
            rdma = pltpu.make_async_remote_copy(
                src_ref=stage_ref.at[s],
                dst_ref=recv_ref.at[s],
                send_sem=send_sems.at[s],
                recv_sem=recv_sems.at[s],
                device_id=(right,),
                device_id_type=pl.DeviceIdType.MESH,
            )
            rdma.start()
            rdma.wait()

            c = lax.rem(my + 2 * N_DEV - 2 - s, N_DEV)
            total = recv_ref[s].astype(jnp.float32) + chunk_partial(c)
            if s < N_DEV - 2:
                stage_ref[s + 1] = total.astype(COMM_DTYPE)
            else:
                out_ref[...] = total * (sx_ref[0] * sw_ref[0])

    return pl.pallas_call(
        body,
        out_shape=jax.ShapeDtypeStruct((m_per, n), jnp.float32),
        in_specs=[
            pl.BlockSpec(memory_space=pltpu.VMEM),
            pl.BlockSpec(memory_space=pltpu.VMEM),
            pl.BlockSpec(memory_space=pltpu.SMEM),
            pl.BlockSpec(memory_space=pltpu.SMEM),
        ],
        out_specs=pl.BlockSpec(memory_space=pltpu.VMEM),
        scratch_shapes=[
            pltpu.VMEM((N_DEV - 1, m_per, n), COMM_DTYPE),
            pltpu.VMEM((N_DEV - 1, m_per, n), COMM_DTYPE),
            pltpu.SemaphoreType.DMA((N_DEV - 1,)),
            pltpu.SemaphoreType.DMA((N_DEV - 1,)),
        ],
        compiler_params=pltpu.CompilerParams(collective_id=0),
    )(x, w_mat, scale_x, scale_w)


# baseline (device time: 176043 ns/iter reference)
import jax
import jax.numpy as jnp
from jax import lax
from jax.experimental import pallas as pl
from jax.experimental.pallas import tpu as pltpu

N_DEV = 4
COMM_DTYPE = jnp.bfloat16


def kernel(x, w_mat, scale_x, scale_w):
    m, _k = x.shape
    _, n = w_mat.shape
    m_per = m // N_DEV

    x = x.astype(jnp.bfloat16)
    w_mat = w_mat.astype(jnp.bfloat16)

    def body(x_ref, w_ref, sx_ref, sw_ref, out_ref,
             stage_ref, recv_ref, send_sems, recv_sems):
        my = lax.axis_index("i")
        left = lax.rem(my + N_DEV - 1, N_DEV)
        right = lax.rem(my + 1, N_DEV)

        barrier_sem = pltpu.get_barrier_semaphore()
        for nbr in (left, right):
            pl.semaphore_signal(
                barrier_sem, inc=1,
                device_id=(nbr,), device_id_type=pl.DeviceIdType.MESH,
            )
        pl.semaphore_wait(barrier_sem, 2)

        def chunk_partial(c):
            xc = x_ref[pl.ds(c * m_per, m_per), :]
            return jnp.dot(xc, w_ref[...], preferred_element_type=jnp.float32)

        c0 = lax.rem(my + N_DEV - 1, N_DEV)
        stage_ref[0] = chunk_partial(c0).astype(COMM_DTYPE)

        for s in range(N_DEV - 1):
            rdma = pltpu.make_async_remote_copy(
                src_ref=stage_ref.at[s],
                dst_ref=recv_ref.at[s],
                send_sem=send_sems.at[s],
                recv_sem=recv_sems.at[s],
                device_id=(right,),
                device_id_type=pl.DeviceIdType.MESH,
            )
            rdma.start()
            rdma.wait()

            c = lax.rem(my + 2 * N_DEV - 2 - s, N_DEV)
            total = recv_ref[s].astype(jnp.float32) + chunk_partial(c)
            if s < N_DEV - 2:
                stage_ref[s + 1] = total.astype(COMM_DTYPE)
            else:
                out_ref[...] = total * (sx_ref[0] * sw_ref[0])

    return pl.pallas_call(
        body,
        out_shape=jax.ShapeDtypeStruct((m_per, n), jnp.float32),
        in_specs=[
            pl.BlockSpec(memory_space=pltpu.VMEM),
            pl.BlockSpec(memory_space=pltpu.VMEM),
            pl.BlockSpec(memory_space=pltpu.SMEM),
            pl.BlockSpec(memory_space=pltpu.SMEM),
        ],
        out_specs=pl.BlockSpec(memory_space=pltpu.VMEM),
        scratch_shapes=[
            pltpu.VMEM((N_DEV - 1, m_per, n), COMM_DTYPE),
            pltpu.VMEM((N_DEV - 1, m_per, n), COMM_DTYPE),
            pltpu.SemaphoreType.DMA((N_DEV - 1,)),
            pltpu.SemaphoreType.DMA((N_DEV - 1,)),
        ],
        compiler_params=pltpu.CompilerParams(collective_id=0),
    )(x, w_mat, scale_x, scale_w)


# device time: 105957 ns/iter; 1.6615x vs baseline; 1.6615x over previous
import jax
import jax.numpy as jnp
from jax import lax
from jax.experimental import pallas as pl
from jax.experimental.pallas import tpu as pltpu

N_DEV = 4
COMM_DTYPE = jnp.bfloat16


def kernel(x, w_mat, scale_x, scale_w):
    m, _k = x.shape
    _, n = w_mat.shape
    m_per = m // N_DEV
    half = n // 2

    x = x.astype(jnp.bfloat16)
    w_mat = w_mat.astype(jnp.bfloat16)

    def body(x_ref, w_ref, sx_ref, sw_ref, out_ref,
             stage_r, stage_l,
             send_sems_r, recv_sems_r, send_sems_l, recv_sems_l):
        my = lax.axis_index("i")
        left = lax.rem(my + N_DEV - 1, N_DEV)
        right = lax.rem(my + 1, N_DEV)

        barrier_sem = pltpu.get_barrier_semaphore()
        for nbr in (left, right):
            pl.semaphore_signal(
                barrier_sem, inc=1,
                device_id=(nbr,), device_id_type=pl.DeviceIdType.MESH,
            )
        pl.semaphore_wait(barrier_sem, 2)

        def partial_r(c):
            xc = x_ref[pl.ds(c * m_per, m_per), :]
            return jnp.dot(xc, w_ref[:, :half],
                           preferred_element_type=jnp.float32)

        def partial_l(c):
            xc = x_ref[pl.ds(c * m_per, m_per), :]
            return jnp.dot(xc, w_ref[:, half:],
                           preferred_element_type=jnp.float32)

        stage_r[0] = partial_r(lax.rem(my + N_DEV - 1, N_DEV)).astype(COMM_DTYPE)
        stage_l[0] = partial_l(lax.rem(my + 1, N_DEV)).astype(COMM_DTYPE)

        scale = sx_ref[0] * sw_ref[0]

        for s in range(N_DEV - 1):
            rdma_r = pltpu.make_async_remote_copy(
                src_ref=stage_r.at[s], dst_ref=stage_r.at[s + 1],
                send_sem=send_sems_r.at[s], recv_sem=recv_sems_r.at[s],
                device_id=(right,), device_id_type=pl.DeviceIdType.MESH,
            )
            rdma_l = pltpu.make_async_remote_copy(
                src_ref=stage_l.at[s], dst_ref=stage_l.at[s + 1],
                send_sem=send_sems_l.at[s], recv_sem=recv_sems_l.at[s],
                device_id=(left,), device_id_type=pl.DeviceIdType.MESH,
            )
            rdma_r.start()
            rdma_l.start()

            cr = lax.rem(my + 2 * N_DEV - 2 - s, N_DEV)
            cl = lax.rem(my + 2 + s, N_DEV)
            pr = partial_r(cr)
            rdma_r.wait()
            tr = stage_r[s + 1].astype(jnp.float32) + pr
            if s < N_DEV - 2:
                stage_r[s + 1] = tr.astype(COMM_DTYPE)
            else:
                out_ref[:, :half] = tr * scale

            pl_ = partial_l(cl)
            rdma_l.wait()
            tl = stage_l[s + 1].astype(jnp.float32) + pl_
            if s < N_DEV - 2:
                stage_l[s + 1] = tl.astype(COMM_DTYPE)
            else:
                out_ref[:, half:] = tl * scale

    return pl.pallas_call(
        body,
        out_shape=jax.ShapeDtypeStruct((m_per, n), jnp.float32),
        in_specs=[
            pl.BlockSpec(memory_space=pltpu.VMEM),
            pl.BlockSpec(memory_space=pltpu.VMEM),
            pl.BlockSpec(memory_space=pltpu.SMEM),
            pl.BlockSpec(memory_space=pltpu.SMEM),
        ],
        out_specs=pl.BlockSpec(memory_space=pltpu.VMEM),
        scratch_shapes=[
            pltpu.VMEM((N_DEV, m_per, half), COMM_DTYPE),
            pltpu.VMEM((N_DEV, m_per, half), COMM_DTYPE),
            pltpu.SemaphoreType.DMA((N_DEV - 1,)),
            pltpu.SemaphoreType.DMA((N_DEV - 1,)),
            pltpu.SemaphoreType.DMA((N_DEV - 1,)),
            pltpu.SemaphoreType.DMA((N_DEV - 1,)),
        ],
        compiler_params=pltpu.CompilerParams(collective_id=0),
    )(x, w_mat, scale_x, scale_w)


# device time: 89398 ns/iter; 1.9692x vs baseline; 1.1852x over previous
import jax
import jax.numpy as jnp
from jax import lax
from jax.experimental import pallas as pl
from jax.experimental.pallas import tpu as pltpu

N_DEV = 4
NSUB = 2
COMM_DTYPE = jnp.bfloat16


def kernel(x, w_mat, scale_x, scale_w):
    m, _k = x.shape
    _, n = w_mat.shape
    m_per = m // N_DEV
    half = n // 2
    sub = half // NSUB

    x = x.astype(jnp.bfloat16)
    w_mat = w_mat.astype(jnp.bfloat16)

    def body(x_ref, w_ref, sx_ref, sw_ref, out_ref,
             stage_r, stage_l,
             send_sems_r, recv_sems_r, send_sems_l, recv_sems_l):
        my = lax.axis_index("i")
        left = lax.rem(my + N_DEV - 1, N_DEV)
        right = lax.rem(my + 1, N_DEV)

        barrier_sem = pltpu.get_barrier_semaphore()
        for nbr in (left, right):
            pl.semaphore_signal(
                barrier_sem, inc=1,
                device_id=(nbr,), device_id_type=pl.DeviceIdType.MESH,
            )
        pl.semaphore_wait(barrier_sem, 2)

        def partial(c, col0):
            xc = x_ref[pl.ds(c * m_per, m_per), :]
            return jnp.dot(xc, w_ref[:, col0:col0 + sub],
                           preferred_element_type=jnp.float32)

        scale = sx_ref[0] * sw_ref[0]

        dirs = [
            (stage_r, send_sems_r, recv_sems_r, right, 0,
             lambda h: lax.rem(my + 2 * N_DEV - 2 - h, N_DEV)),
            (stage_l, send_sems_l, recv_sems_l, left, half,
             lambda h: lax.rem(my + 2 + h, N_DEV)),
        ]

        def make_rdma(h, d, b):
            stage, ssems, rsems, nbr, col0, _ = dirs[d]
            return pltpu.make_async_remote_copy(
                src_ref=stage.at[h, :, pl.ds(b * sub, sub)],
                dst_ref=stage.at[h + 1, :, pl.ds(b * sub, sub)],
                send_sem=ssems.at[h, b],
                recv_sem=rsems.at[h, b],
                device_id=(nbr,), device_id_type=pl.DeviceIdType.MESH,
            )

        sends = {}

        for b in range(NSUB):
            for d in range(2):
                stage, _, _, nbr, col0, chunk_at = dirs[d]
                c0 = chunk_at(-1)
                stage[0, :, pl.ds(b * sub, sub)] = (
                    partial(c0, col0 + b * sub).astype(COMM_DTYPE))
                r = make_rdma(0, d, b)
                r.start()
                sends[(0, d, b)] = r

        for h in range(N_DEV - 1):
            last = h == N_DEV - 2
            for b in range(NSUB):
                for d in range(2):
                    stage, _, _, nbr, col0, chunk_at = dirs[d]
                    c = chunk_at(h)
                    p = partial(c, col0 + b * sub)
                    sends[(h, d, b)].wait_recv()
                    t = stage[h + 1, :, pl.ds(b * sub, sub)].astype(
                        jnp.float32) + p
                    if not last:
                        stage[h + 1, :, pl.ds(b * sub, sub)] = (
                            t.astype(COMM_DTYPE))
                        r = make_rdma(h + 1, d, b)
                        r.start()
                        sends[(h + 1, d, b)] = r
                    else:
                        out_ref[:, pl.ds(col0 + b * sub, sub)] = t * scale

        for r in sends.values():
            r.wait_send()

    return pl.pallas_call(
        body,
        out_shape=jax.ShapeDtypeStruct((m_per, n), jnp.float32),
        in_specs=[
            pl.BlockSpec(memory_space=pltpu.VMEM),
            pl.BlockSpec(memory_space=pltpu.VMEM),
            pl.BlockSpec(memory_space=pltpu.SMEM),
            pl.BlockSpec(memory_space=pltpu.SMEM),
        ],
        out_specs=pl.BlockSpec(memory_space=pltpu.VMEM),
        scratch_shapes=[
            pltpu.VMEM((N_DEV, m_per, half), COMM_DTYPE),
            pltpu.VMEM((N_DEV, m_per, half), COMM_DTYPE),
            pltpu.SemaphoreType.DMA((N_DEV - 1, NSUB)),
            pltpu.SemaphoreType.DMA((N_DEV - 1, NSUB)),
            pltpu.SemaphoreType.DMA((N_DEV - 1, NSUB)),
            pltpu.SemaphoreType.DMA((N_DEV - 1, NSUB)),
        ],
        compiler_params=pltpu.CompilerParams(collective_id=0),
    )(x, w_mat, scale_x, scale_w)
